# Initial kernel scaffold; baseline (speedup 1.0000x reference)
import jax
import jax.numpy as jnp
from jax.experimental import pallas as pl


def kernel(x, edge_index, batch, g0_w1, g0_b1, g0_w2, g0_b2, g0_gamma, g0_beta, g1_w1, g1_b1, g1_w2, g1_b2, g1_gamma, g1_beta, g2_w1, g2_b1, g2_w2, g2_b2, g2_gamma, g2_beta, h1_w, h1_b, h1_gamma, h1_beta, h2_w, h2_b, h2_gamma, h2_beta, out_w, out_b):
    raise NotImplementedError("write your pallas kernel here")


# speedup vs baseline: 6.0866x; 6.0866x over previous
import functools

import jax
import jax.numpy as jnp
from jax import lax
from jax.experimental import pallas as pl
from jax.experimental.pallas import tpu as pltpu
from jax.experimental.pallas import tpu_sc as plsc

N = 10000
E = 320000
D = 128
G = 64

NC = 2
NS = 16
NW = NC * NS
EPW = E // NW
CHUNK = 80
NCH = EPW // CHUNK
RPW = 624
TAIL = N - NS * RPW
TAIL_OFF = NS * RPW


def _sc_aggregate(h, src_r, dst_r):
    mesh = plsc.VectorSubcoreMesh(core_axis_name="c", subcore_axis_name="s")

    @functools.partial(
        pl.kernel,
        out_type=jax.ShapeDtypeStruct((NC, N, D), jnp.float32),
        mesh=mesh,
        scratch_types=[
            pltpu.VMEM_SHARED((N, D), jnp.float32),
            pltpu.VMEM((NCH, CHUNK), jnp.int32),
            pltpu.VMEM((NCH, CHUNK), jnp.int32),
            pltpu.VMEM((CHUNK, D), jnp.float32),
            pltpu.SemaphoreType.DMA,
        ],
    )
    def k(h_hbm, src_hbm, dst_hbm, out_hbm, acc_sh, src_v, dst_v, rows_v, sem):
        c = lax.axis_index("c")
        s = lax.axis_index("s")
        wid = c * NS + s
        base = s * RPW
        pltpu.sync_copy(h_hbm.at[pl.ds(base, RPW)], acc_sh.at[pl.ds(base, RPW)])

        @pl.when(s == 0)
        def _():
            pltpu.sync_copy(h_hbm.at[pl.ds(TAIL_OFF, TAIL)],
                            acc_sh.at[pl.ds(TAIL_OFF, TAIL)])

        pltpu.sync_copy(src_hbm.at[wid], src_v)
        pltpu.sync_copy(dst_hbm.at[wid], dst_v)
        plsc.subcore_barrier()

        def body(j, carry):
            pltpu.async_copy(h_hbm.at[src_v.at[j]], rows_v, sem).wait()
            pltpu.sync_copy(rows_v, acc_sh.at[dst_v.at[j]], add=True)
            return carry

        lax.fori_loop(0, NCH, body, 0)
        plsc.subcore_barrier()
        pltpu.sync_copy(acc_sh.at[pl.ds(base, RPW)],
                        out_hbm.at[c, pl.ds(base, RPW)])

        @pl.when(s == 0)
        def _():
            pltpu.sync_copy(acc_sh.at[pl.ds(TAIL_OFF, TAIL)],
                            out_hbm.at[c, pl.ds(TAIL_OFF, TAIL)])

    return k(h, src_r, dst_r)


MLP_BLK = 400


def _mlp_pass(a, h, w1, b1, w2, b2):
    def kern(a_ref, h_ref, w1_ref, b1_ref, w2_ref, b2_ref, t_ref, st_ref):
        i = pl.program_id(0)
        m = a_ref[0] + a_ref[1] - h_ref[...]
        t = jnp.maximum(
            jnp.dot(m.astype(jnp.bfloat16),
                    w1_ref[...].astype(jnp.bfloat16),
                    preferred_element_type=jnp.float32)
            + b1_ref[...], 0.0)
        t = (jnp.dot(t.astype(jnp.bfloat16),
                     w2_ref[...].astype(jnp.bfloat16),
                     preferred_element_type=jnp.float32)
             + b2_ref[...])
        t_ref[...] = t

        @pl.when(i == 0)
        def _():
            st_ref[...] = jnp.zeros_like(st_ref)

        su = jnp.sum(t, axis=0, keepdims=True)
        sq = jnp.sum(t * t, axis=0, keepdims=True)
        st_ref[...] += jnp.concatenate([su, sq], axis=0)

    grid = N // MLP_BLK
    return pl.pallas_call(
        kern,
        grid=(grid,),
        in_specs=[
            pl.BlockSpec((NC, MLP_BLK, D), lambda i: (0, i, 0)),
            pl.BlockSpec((MLP_BLK, D), lambda i: (i, 0)),
            pl.BlockSpec((D, D), lambda i: (0, 0)),
            pl.BlockSpec((1, D), lambda i: (0, 0)),
            pl.BlockSpec((D, D), lambda i: (0, 0)),
            pl.BlockSpec((1, D), lambda i: (0, 0)),
        ],
        out_specs=[
            pl.BlockSpec((MLP_BLK, D), lambda i: (i, 0)),
            pl.BlockSpec((2, D), lambda i: (0, 0)),
        ],
        out_shape=[
            jax.ShapeDtypeStruct((N, D), jnp.float32),
            jax.ShapeDtypeStruct((2, D), jnp.float32),
        ],
    )(a, h, w1, b1.reshape(1, D), w2, b2.reshape(1, D))


BN_BLK = 2000


def _bn_relu(t, st, gamma, beta):
    def kern(t_ref, st_ref, g_ref, b_ref, o_ref):
        mean = st_ref[0:1, :] * (1.0 / N)
        var = st_ref[1:2, :] * (1.0 / N) - mean * mean
        scale = g_ref[...] * lax.rsqrt(var + 1e-5)
        shift = b_ref[...] - mean * scale
        o_ref[...] = jnp.maximum(t_ref[...] * scale + shift, 0.0)

    return pl.pallas_call(
        kern,
        grid=(N // BN_BLK,),
        in_specs=[
            pl.BlockSpec((BN_BLK, D), lambda i: (i, 0)),
            pl.BlockSpec((2, D), lambda i: (0, 0)),
            pl.BlockSpec((1, D), lambda i: (0, 0)),
            pl.BlockSpec((1, D), lambda i: (0, 0)),
        ],
        out_specs=pl.BlockSpec((BN_BLK, D), lambda i: (i, 0)),
        out_shape=jax.ShapeDtypeStruct((N, D), jnp.float32),
    )(t, st, gamma.reshape(1, D), beta.reshape(1, D))


POOL_BLK = 400


def _pool(h, batch3d):
    def kern(b_ref, h_ref, s_ref, c_ref):
        i = pl.program_id(0)

        @pl.when(i == 0)
        def _():
            s_ref[...] = jnp.zeros_like(s_ref)
            c_ref[...] = jnp.zeros_like(c_ref)

        seg = b_ref[0]
        onehot = (lax.broadcasted_iota(jnp.int32, (G, POOL_BLK), 0) == seg
                  ).astype(jnp.float32)
        s_ref[...] += jnp.dot(onehot, h_ref[...],
                              preferred_element_type=jnp.float32,
                              precision=lax.Precision.HIGHEST)
        c_ref[...] += jnp.broadcast_to(
            jnp.sum(onehot, axis=1, keepdims=True), (G, D))

    grid = N // POOL_BLK
    return pl.pallas_call(
        kern,
        grid=(grid,),
        in_specs=[
            pl.BlockSpec((1, 1, POOL_BLK), lambda i: (i, 0, 0)),
            pl.BlockSpec((POOL_BLK, D), lambda i: (i, 0)),
        ],
        out_specs=[
            pl.BlockSpec((G, D), lambda i: (0, 0)),
            pl.BlockSpec((G, D), lambda i: (0, 0)),
        ],
        out_shape=[
            jax.ShapeDtypeStruct((G, D), jnp.float32),
            jax.ShapeDtypeStruct((G, D), jnp.float32),
        ],
    )(batch3d, h)


def _head(s, cnt, h1_w, h1_b, h1_gamma, h1_beta,
          h2_w, h2_b, h2_gamma, h2_beta, out_w, out_b):
    C = out_w.shape[1]
    H2 = h2_w.shape[1]

    def kern(s_ref, c_ref, w1_ref, b1_ref, g1_ref, be1_ref,
             w2_ref, b2_ref, g2_ref, be2_ref, wo_ref, bo_ref, o_ref):
        p = s_ref[...] / jnp.maximum(c_ref[...], 1.0)
        p = jnp.dot(p.astype(jnp.bfloat16),
                    w1_ref[...].astype(jnp.bfloat16),
                    preferred_element_type=jnp.float32) + b1_ref[...]
        mean = jnp.mean(p, axis=0, keepdims=True)
        var = jnp.mean(p * p, axis=0, keepdims=True) - mean * mean
        p = jnp.maximum(
            g1_ref[...] * (p - mean) * lax.rsqrt(var + 1e-5) + be1_ref[...],
            0.0)
        p = jnp.dot(p.astype(jnp.bfloat16),
                    w2_ref[...].astype(jnp.bfloat16),
                    preferred_element_type=jnp.float32) + b2_ref[...]
        mean = jnp.mean(p, axis=0, keepdims=True)
        var = jnp.mean(p * p, axis=0, keepdims=True) - mean * mean
        p = jnp.maximum(
            g2_ref[...] * (p - mean) * lax.rsqrt(var + 1e-5) + be2_ref[...],
            0.0)
        o_ref[...] = jnp.dot(p.astype(jnp.bfloat16),
                             wo_ref[...].astype(jnp.bfloat16),
                             preferred_element_type=jnp.float32) + bo_ref[...]

    return pl.pallas_call(
        kern,
        out_shape=jax.ShapeDtypeStruct((G, C), jnp.float32),
    )(s, cnt, h1_w, h1_b.reshape(1, D), h1_gamma.reshape(1, D),
      h1_beta.reshape(1, D), h2_w, h2_b.reshape(1, H2),
      h2_gamma.reshape(1, H2), h2_beta.reshape(1, H2), out_w,
      out_b.reshape(1, C))


def kernel(x, edge_index, batch,
           g0_w1, g0_b1, g0_w2, g0_b2, g0_gamma, g0_beta,
           g1_w1, g1_b1, g1_w2, g1_b2, g1_gamma, g1_beta,
           g2_w1, g2_b1, g2_w2, g2_b2, g2_gamma, g2_beta,
           h1_w, h1_b, h1_gamma, h1_beta,
           h2_w, h2_b, h2_gamma, h2_beta,
           out_w, out_b):
    src_r = edge_index[0].reshape(NW, NCH, CHUNK)
    dst_r = edge_index[1].reshape(NW, NCH, CHUNK)
    batch3d = batch.reshape(N // POOL_BLK, 1, POOL_BLK)

    h = x
    for (w1, b1, w2, b2, gamma, beta) in (
            (g0_w1, g0_b1, g0_w2, g0_b2, g0_gamma, g0_beta),
            (g1_w1, g1_b1, g1_w2, g1_b2, g1_gamma, g1_beta),
            (g2_w1, g2_b1, g2_w2, g2_b2, g2_gamma, g2_beta)):
        a = _sc_aggregate(h, src_r, dst_r)
        t, st = _mlp_pass(a, h, w1, b1, w2, b2)
        h = _bn_relu(t, st, gamma, beta)

    s, cnt = _pool(h, batch3d)
    return _head(s, cnt, h1_w, h1_b, h1_gamma, h1_beta,
                 h2_w, h2_b, h2_gamma, h2_beta, out_w, out_b)


# speedup vs baseline: 8.9304x; 1.4672x over previous
import functools

import jax
import jax.numpy as jnp
from jax import lax
from jax.experimental import pallas as pl
from jax.experimental.pallas import tpu as pltpu
from jax.experimental.pallas import tpu_sc as plsc

N = 10000
E = 320000
D = 128
G = 64

NC = 2
NS = 16
NW = NC * NS
EPW = E // NW
CHUNK = 80
NCH = EPW // CHUNK
QCH = 32
STAGES = [32, 32, 32, 29]
RPW = 624
TAIL = N - NS * RPW
TAIL_OFF = NS * RPW


def _sc_aggregate(h, src_r, dst_r):
    mesh = plsc.VectorSubcoreMesh(core_axis_name="c", subcore_axis_name="s")

    @functools.partial(
        pl.kernel,
        out_type=jax.ShapeDtypeStruct((NC, N, D), jnp.float32),
        mesh=mesh,
        scratch_types=[
            pltpu.VMEM_SHARED((N, D), jnp.float32),
            pltpu.VMEM((QCH, CHUNK), jnp.int32),
            pltpu.VMEM((QCH, CHUNK), jnp.int32),
            pltpu.VMEM((CHUNK, D), jnp.float32),
            pltpu.VMEM((CHUNK, D), jnp.float32),
            pltpu.SemaphoreType.DMA,
            pltpu.SemaphoreType.DMA,
        ],
    )
    def k(h_hbm, src_hbm, dst_hbm, out_hbm, acc_sh, src_v, dst_v,
          rows_a, rows_b, sem_a, sem_b):
        c = lax.axis_index("c")
        s = lax.axis_index("s")
        wid = c * NS + s
        base = s * RPW
        pltpu.sync_copy(h_hbm.at[pl.ds(base, RPW)], acc_sh.at[pl.ds(base, RPW)])

        @pl.when(s == 0)
        def _():
            pltpu.sync_copy(h_hbm.at[pl.ds(TAIL_OFF, TAIL)],
                            acc_sh.at[pl.ds(TAIL_OFF, TAIL)])

        plsc.subcore_barrier()

        off = 0
        for n in STAGES:
            pltpu.sync_copy(src_hbm.at[wid, pl.ds(off, n)],
                            src_v.at[pl.ds(0, n)])
            pltpu.sync_copy(dst_hbm.at[wid, pl.ds(off, n)],
                            dst_v.at[pl.ds(0, n)])
            pltpu.async_copy(h_hbm.at[src_v.at[0]], rows_a, sem_a)

            def pair(i, carry, n=n):
                j = 2 * i
                pltpu.async_copy(h_hbm.at[src_v.at[j + 1]], rows_b, sem_b)
                pltpu.make_async_copy(h_hbm.at[src_v.at[j]], rows_a,
                                      sem_a).wait()
                pltpu.sync_copy(rows_a, acc_sh.at[dst_v.at[j]], add=True)

                @pl.when(j + 2 < n)
                def _():
                    pltpu.async_copy(h_hbm.at[src_v.at[j + 2]], rows_a, sem_a)

                pltpu.make_async_copy(h_hbm.at[src_v.at[j + 1]], rows_b,
                                      sem_b).wait()
                pltpu.sync_copy(rows_b, acc_sh.at[dst_v.at[j + 1]], add=True)
                return carry

            lax.fori_loop(0, n // 2, pair, 0)
            if n % 2:
                pltpu.make_async_copy(h_hbm.at[src_v.at[n - 1]], rows_a,
                                      sem_a).wait()
                pltpu.sync_copy(rows_a, acc_sh.at[dst_v.at[n - 1]], add=True)
            off += n
        plsc.subcore_barrier()
        pltpu.sync_copy(acc_sh.at[pl.ds(base, RPW)],
                        out_hbm.at[c, pl.ds(base, RPW)])

        @pl.when(s == 0)
        def _():
            pltpu.sync_copy(acc_sh.at[pl.ds(TAIL_OFF, TAIL)],
                            out_hbm.at[c, pl.ds(TAIL_OFF, TAIL)])

    return k(h, src_r, dst_r)


MLP_BLK = 400


def _mlp_pass(a, h, w1, b1, w2, b2):
    def kern(a_ref, h_ref, w1_ref, b1_ref, w2_ref, b2_ref, t_ref, st_ref):
        i = pl.program_id(0)
        m = a_ref[0] + a_ref[1] - h_ref[...]
        t = jnp.maximum(
            jnp.dot(m.astype(jnp.bfloat16),
                    w1_ref[...].astype(jnp.bfloat16),
                    preferred_element_type=jnp.float32)
            + b1_ref[...], 0.0)
        t = (jnp.dot(t.astype(jnp.bfloat16),
                     w2_ref[...].astype(jnp.bfloat16),
                     preferred_element_type=jnp.float32)
             + b2_ref[...])
        t_ref[...] = t

        @pl.when(i == 0)
        def _():
            st_ref[...] = jnp.zeros_like(st_ref)

        su = jnp.sum(t, axis=0, keepdims=True)
        sq = jnp.sum(t * t, axis=0, keepdims=True)
        st_ref[...] += jnp.concatenate([su, sq], axis=0)

    grid = N // MLP_BLK
    return pl.pallas_call(
        kern,
        grid=(grid,),
        in_specs=[
            pl.BlockSpec((NC, MLP_BLK, D), lambda i: (0, i, 0)),
            pl.BlockSpec((MLP_BLK, D), lambda i: (i, 0)),
            pl.BlockSpec((D, D), lambda i: (0, 0)),
            pl.BlockSpec((1, D), lambda i: (0, 0)),
            pl.BlockSpec((D, D), lambda i: (0, 0)),
            pl.BlockSpec((1, D), lambda i: (0, 0)),
        ],
        out_specs=[
            pl.BlockSpec((MLP_BLK, D), lambda i: (i, 0)),
            pl.BlockSpec((2, D), lambda i: (0, 0)),
        ],
        out_shape=[
            jax.ShapeDtypeStruct((N, D), jnp.float32),
            jax.ShapeDtypeStruct((2, D), jnp.float32),
        ],
    )(a, h, w1, b1.reshape(1, D), w2, b2.reshape(1, D))


BN_BLK = 2000


def _bn_relu(t, st, gamma, beta):
    def kern(t_ref, st_ref, g_ref, b_ref, o_ref):
        mean = st_ref[0:1, :] * (1.0 / N)
        var = st_ref[1:2, :] * (1.0 / N) - mean * mean
        scale = g_ref[...] * lax.rsqrt(var + 1e-5)
        shift = b_ref[...] - mean * scale
        o_ref[...] = jnp.maximum(t_ref[...] * scale + shift, 0.0)

    return pl.pallas_call(
        kern,
        grid=(N // BN_BLK,),
        in_specs=[
            pl.BlockSpec((BN_BLK, D), lambda i: (i, 0)),
            pl.BlockSpec((2, D), lambda i: (0, 0)),
            pl.BlockSpec((1, D), lambda i: (0, 0)),
            pl.BlockSpec((1, D), lambda i: (0, 0)),
        ],
        out_specs=pl.BlockSpec((BN_BLK, D), lambda i: (i, 0)),
        out_shape=jax.ShapeDtypeStruct((N, D), jnp.float32),
    )(t, st, gamma.reshape(1, D), beta.reshape(1, D))


POOL_BLK = 400


def _pool(h, batch3d):
    def kern(b_ref, h_ref, s_ref, c_ref):
        i = pl.program_id(0)

        @pl.when(i == 0)
        def _():
            s_ref[...] = jnp.zeros_like(s_ref)
            c_ref[...] = jnp.zeros_like(c_ref)

        seg = b_ref[0]
        onehot = (lax.broadcasted_iota(jnp.int32, (G, POOL_BLK), 0) == seg
                  ).astype(jnp.float32)
        s_ref[...] += jnp.dot(onehot, h_ref[...],
                              preferred_element_type=jnp.float32,
                              precision=lax.Precision.HIGHEST)
        c_ref[...] += jnp.broadcast_to(
            jnp.sum(onehot, axis=1, keepdims=True), (G, D))

    grid = N // POOL_BLK
    return pl.pallas_call(
        kern,
        grid=(grid,),
        in_specs=[
            pl.BlockSpec((1, 1, POOL_BLK), lambda i: (i, 0, 0)),
            pl.BlockSpec((POOL_BLK, D), lambda i: (i, 0)),
        ],
        out_specs=[
            pl.BlockSpec((G, D), lambda i: (0, 0)),
            pl.BlockSpec((G, D), lambda i: (0, 0)),
        ],
        out_shape=[
            jax.ShapeDtypeStruct((G, D), jnp.float32),
            jax.ShapeDtypeStruct((G, D), jnp.float32),
        ],
    )(batch3d, h)


def _head(s, cnt, h1_w, h1_b, h1_gamma, h1_beta,
          h2_w, h2_b, h2_gamma, h2_beta, out_w, out_b):
    C = out_w.shape[1]
    H2 = h2_w.shape[1]

    def kern(s_ref, c_ref, w1_ref, b1_ref, g1_ref, be1_ref,
             w2_ref, b2_ref, g2_ref, be2_ref, wo_ref, bo_ref, o_ref):
        p = s_ref[...] / jnp.maximum(c_ref[...], 1.0)
        p = jnp.dot(p.astype(jnp.bfloat16),
                    w1_ref[...].astype(jnp.bfloat16),
                    preferred_element_type=jnp.float32) + b1_ref[...]
        mean = jnp.mean(p, axis=0, keepdims=True)
        var = jnp.mean(p * p, axis=0, keepdims=True) - mean * mean
        p = jnp.maximum(
            g1_ref[...] * (p - mean) * lax.rsqrt(var + 1e-5) + be1_ref[...],
            0.0)
        p = jnp.dot(p.astype(jnp.bfloat16),
                    w2_ref[...].astype(jnp.bfloat16),
                    preferred_element_type=jnp.float32) + b2_ref[...]
        mean = jnp.mean(p, axis=0, keepdims=True)
        var = jnp.mean(p * p, axis=0, keepdims=True) - mean * mean
        p = jnp.maximum(
            g2_ref[...] * (p - mean) * lax.rsqrt(var + 1e-5) + be2_ref[...],
            0.0)
        o_ref[...] = jnp.dot(p.astype(jnp.bfloat16),
                             wo_ref[...].astype(jnp.bfloat16),
                             preferred_element_type=jnp.float32) + bo_ref[...]

    return pl.pallas_call(
        kern,
        out_shape=jax.ShapeDtypeStruct((G, C), jnp.float32),
    )(s, cnt, h1_w, h1_b.reshape(1, D), h1_gamma.reshape(1, D),
      h1_beta.reshape(1, D), h2_w, h2_b.reshape(1, H2),
      h2_gamma.reshape(1, H2), h2_beta.reshape(1, H2), out_w,
      out_b.reshape(1, C))


def kernel(x, edge_index, batch,
           g0_w1, g0_b1, g0_w2, g0_b2, g0_gamma, g0_beta,
           g1_w1, g1_b1, g1_w2, g1_b2, g1_gamma, g1_beta,
           g2_w1, g2_b1, g2_w2, g2_b2, g2_gamma, g2_beta,
           h1_w, h1_b, h1_gamma, h1_beta,
           h2_w, h2_b, h2_gamma, h2_beta,
           out_w, out_b):
    src_r = edge_index[0].reshape(NW, NCH, CHUNK)
    dst_r = edge_index[1].reshape(NW, NCH, CHUNK)
    batch3d = batch.reshape(N // POOL_BLK, 1, POOL_BLK)

    h = x
    for (w1, b1, w2, b2, gamma, beta) in (
            (g0_w1, g0_b1, g0_w2, g0_b2, g0_gamma, g0_beta),
            (g1_w1, g1_b1, g1_w2, g1_b2, g1_gamma, g1_beta),
            (g2_w1, g2_b1, g2_w2, g2_b2, g2_gamma, g2_beta)):
        a = _sc_aggregate(h, src_r, dst_r)
        t, st = _mlp_pass(a, h, w1, b1, w2, b2)
        h = _bn_relu(t, st, gamma, beta)

    s, cnt = _pool(h, batch3d)
    return _head(s, cnt, h1_w, h1_b, h1_gamma, h1_beta,
                 h2_w, h2_b, h2_gamma, h2_beta, out_w, out_b)
